# baseline (speedup 1.0000x reference)
import functools

import jax
import jax.numpy as jnp
from jax import lax
from jax.experimental import pallas as pl
from jax.experimental.pallas import tpu as pltpu
from jax.experimental.pallas import tpu_sc as plsc

N = 10000
E = 320000
F_IN = 128
H = 64
ED = 4
L = 4
OUT = 5

WT = 128
WO = 128

BN = 2000
BE = 2000


def _silu(v):
    return v / (1.0 + jnp.exp(-v))


def _embed_body(x_ref, p_ref, w_ref, b_ref, t_ref):
    h = jnp.dot(x_ref[...], w_ref[...], preferred_element_type=jnp.float32)
    h = h + b_ref[...]
    p = p_ref[...]
    p2 = jnp.sum(p * p, axis=1, keepdims=True)
    z = jnp.zeros((h.shape[0], WT - H - 3), jnp.float32)
    t_ref[...] = jnp.concatenate([h, p, p2, z], axis=1)


def _embed_call(x, pos, W_embed, b_embed):
    return pl.pallas_call(
        _embed_body,
        grid=(N // BN,),
        in_specs=[
            pl.BlockSpec((BN, F_IN), lambda i: (i, 0)),
            pl.BlockSpec((BN, 2), lambda i: (i, 0)),
            pl.BlockSpec((F_IN, H), lambda i: (0, 0)),
            pl.BlockSpec((1, H), lambda i: (0, 0)),
        ],
        out_specs=pl.BlockSpec((BN, WT), lambda i: (i, 0)),
        out_shape=jax.ShapeDtypeStruct((N, WT), jnp.float32),
    )(x, pos, W_embed, b_embed.reshape(1, H))


def _edge_body(ti_ref, tj_ref, ea_ref, wa_ref, wb_ref, vb_ref, wc_ref,
               be1_ref, we2_ref, be2_ref, wc1_ref, bc1_ref, wc28_ref, bc2_ref,
               out_ref):
    ti = ti_ref[...]
    tj = tj_ref[...]
    qi = ti[:, H:H + 8]
    qj = tj[:, H:H + 8]
    qq = qi * qj

    pre = jnp.dot(ti, wa_ref[...], preferred_element_type=jnp.float32)
    pre = pre + jnp.dot(tj, wb_ref[...], preferred_element_type=jnp.float32)
    pre = pre + jnp.dot(qq, vb_ref[...], preferred_element_type=jnp.float32)
    pre = pre + jnp.dot(ea_ref[...], wc_ref[...], preferred_element_type=jnp.float32)
    pre = pre + be1_ref[...]
    m = _silu(pre)
    m = _silu(jnp.dot(m, we2_ref[...], preferred_element_type=jnp.float32) + be2_ref[...])
    c1 = _silu(jnp.dot(m, wc1_ref[...], preferred_element_type=jnp.float32) + bc1_ref[...])
    cw8 = jnp.dot(c1, wc28_ref[...], preferred_element_type=jnp.float32) + bc2_ref[0, 0]

    out_ref[:, :H] = m
    out_ref[:, H:H + 8] = (qi - qj) * cw8
    out_ref[:, H + 8:H + 16] = jnp.ones((ti.shape[0], 8), jnp.float32)


def _edge_call(Ti, Tj, edge_attr, We1, be1, We2, be2, Wc1, bc1, Wc2, bc2):
    wd = We1[2 * H:2 * H + 1]
    z2 = jnp.zeros((2, H), jnp.float32)
    z61 = jnp.zeros((WT - H - 3, H), jnp.float32)
    wa = jnp.concatenate([We1[:H], z2, wd, z61], axis=0)
    wb = jnp.concatenate([We1[H:2 * H], z2, wd, z61], axis=0)
    vb = jnp.concatenate([-2.0 * wd, -2.0 * wd,
                          jnp.zeros((6, H), jnp.float32)], axis=0)
    wc = We1[2 * H + 1:]
    wc28 = jnp.tile(Wc2, (1, 8))
    full = lambda shape: pl.BlockSpec(shape, lambda i: (0, 0))
    return pl.pallas_call(
        _edge_body,
        grid=(EC // BE,),
        in_specs=[
            pl.BlockSpec((BE, WT), lambda i: (i, 0)),
            pl.BlockSpec((BE, WT), lambda i: (i, 0)),
            pl.BlockSpec((BE, ED), lambda i: (i, 0)),
            full((WT, H)), full((WT, H)), full((8, H)), full((ED, H)),
            full((1, H)), full((H, H)), full((1, H)),
            full((H, H)), full((1, H)), full((H, 8)), full((1, 1)),
        ],
        out_specs=pl.BlockSpec((BE, WO), lambda i: (i, 0)),
        out_shape=jax.ShapeDtypeStruct((EC, WO), jnp.float32),
    )(Ti, Tj, edge_attr, wa, wb, vb, wc, be1.reshape(1, H), We2,
      be2.reshape(1, H), Wc1, bc1.reshape(1, H), wc28, bc2.reshape(1, 1))


def _node_body(t_ref, a0_ref, a1_ref, a2_ref, a3_ref, u_ref, wn1a_ref,
               wn1b_ref, wn1c_ref, bn1_ref, wn2_ref, bn2_ref, tout_ref):
    t = t_ref[...]
    h = t[:, :H]
    p = t[:, H:H + 2]
    acc = (a0_ref[...] + a1_ref[...]) + (a2_ref[...] + a3_ref[...])
    agg = acc[:, :H]
    sp = acc[:, H:H + 2]
    deg = acc[:, H + 8:H + 9]
    p_new = p + sp / (deg + 1.0)

    ub = u_ref[0, 0] * wn1c_ref[0:1, :] + u_ref[0, 1] * wn1c_ref[1:2, :]
    pre = jnp.dot(h, wn1a_ref[...], preferred_element_type=jnp.float32)
    pre = pre + jnp.dot(agg, wn1b_ref[...], preferred_element_type=jnp.float32)
    pre = pre + ub + bn1_ref[...]
    hn = _silu(pre)
    hn = jnp.dot(hn, wn2_ref[...], preferred_element_type=jnp.float32) + bn2_ref[...]
    h_new = h + hn
    p2 = jnp.sum(p_new * p_new, axis=1, keepdims=True)
    z = jnp.zeros((h_new.shape[0], WT - H - 3), jnp.float32)
    tout_ref[...] = jnp.concatenate([h_new, p_new, p2, z], axis=1)


def _node_call(T, accs, u, Wn1, bn1, Wn2, bn2):
    wn1a = Wn1[:H]
    wn1b = Wn1[H:2 * H]
    wn1c = Wn1[2 * H:]
    full = lambda shape: pl.BlockSpec(shape, lambda i: (0, 0))
    acc_spec = pl.BlockSpec((BN, WO), lambda i: (i, 0))
    return pl.pallas_call(
        _node_body,
        grid=(N // BN,),
        in_specs=[
            pl.BlockSpec((BN, WT), lambda i: (i, 0)),
            acc_spec, acc_spec, acc_spec, acc_spec,
            full((1, 2)),
            full((H, H)), full((H, H)), full((2, H)),
            full((1, H)), full((H, H)), full((1, H)),
        ],
        out_specs=pl.BlockSpec((BN, WT), lambda i: (i, 0)),
        out_shape=jax.ShapeDtypeStruct((N, WT), jnp.float32),
    )(T, accs[0], accs[1], accs[2], accs[3], u, wn1a, wn1b, wn1c,
      bn1.reshape(1, H), Wn2, bn2.reshape(1, H))


def _head_body(t_ref, w_ref, b_ref, o_ref):
    h = t_ref[:, :H]
    o_ref[...] = jnp.dot(h, w_ref[...], preferred_element_type=jnp.float32) + b_ref[...]


def _head_call(T, W_out, b_out):
    return pl.pallas_call(
        _head_body,
        grid=(N // BN,),
        in_specs=[
            pl.BlockSpec((BN, WT), lambda i: (i, 0)),
            pl.BlockSpec((H, OUT), lambda i: (0, 0)),
            pl.BlockSpec((1, OUT), lambda i: (0, 0)),
        ],
        out_specs=pl.BlockSpec((BN, OUT), lambda i: (i, 0)),
        out_shape=jax.ShapeDtypeStruct((N, OUT), jnp.float32),
    )(T, W_out, b_out.reshape(1, OUT))


NC = 2
NS = 16
NW = NC * NS
NCHUNK = 2
EC = E // NCHUNK
EPW = EC // NW
GCH = 64
GNCH = (EPW + GCH - 1) // GCH
TSR = 640
NP = 10240
RPS = NP // NS


def _chunk_off(i):
    return pl.multiple_of(jnp.minimum(i * GCH, EPW - GCH), 8)


def _s_off(i):
    return pl.multiple_of(jnp.minimum(i * SCH, EPW - SCH), 8)


def _gather_body(t_hbm, dst2_hbm, src2_hbm, ti_hbm, tj_hbm,
                 idxd_v, idxs_v, bufd, bufs, tab_sh, semd, sems, semw):
    c = lax.axis_index("c")
    s = lax.axis_index("s")
    wid = s * NC + c
    base = wid * EPW
    r0 = pl.multiple_of(jnp.minimum(s * TSR, N - TSR), 8)
    pltpu.sync_copy(t_hbm.at[pl.ds(r0, TSR)], tab_sh.at[pl.ds(r0, TSR)])
    pltpu.sync_copy(dst2_hbm.at[wid], idxd_v)
    pltpu.sync_copy(src2_hbm.at[wid], idxs_v)
    plsc.subcore_barrier()

    def start_gather(i, b):
        off = _chunk_off(i)
        pltpu.async_copy(tab_sh.at[idxd_v.at[pl.ds(off, GCH)]], bufd.at[b], semd)
        pltpu.async_copy(tab_sh.at[idxs_v.at[pl.ds(off, GCH)]], bufs.at[b], sems)

    def wait_gather(i, b):
        off = _chunk_off(i)
        pltpu.make_async_copy(tab_sh.at[idxd_v.at[pl.ds(off, GCH)]], bufd.at[b], semd).wait()
        pltpu.make_async_copy(tab_sh.at[idxs_v.at[pl.ds(off, GCH)]], bufs.at[b], sems).wait()

    def start_wb(i, b):
        off = pl.multiple_of(base, 8) + _chunk_off(i)
        pltpu.async_copy(bufd.at[b], ti_hbm.at[pl.ds(off, GCH)], semw)
        pltpu.async_copy(bufs.at[b], tj_hbm.at[pl.ds(off, GCH)], semw)

    def wait_wb(i, b):
        off = pl.multiple_of(base, 8) + _chunk_off(i)
        pltpu.make_async_copy(bufd.at[b], ti_hbm.at[pl.ds(off, GCH)], semw).wait()
        pltpu.make_async_copy(bufs.at[b], tj_hbm.at[pl.ds(off, GCH)], semw).wait()

    start_gather(0, 0)

    def body(i, carry):
        b = lax.rem(i, 2)

        @pl.when(i >= 1)
        def _():
            wait_wb(i - 1, lax.rem(i + 1, 2))

        @pl.when(i + 1 < GNCH)
        def _():
            start_gather(i + 1, lax.rem(i + 1, 2))

        wait_gather(i, b)
        start_wb(i, b)
        return carry

    lax.fori_loop(0, GNCH, body, 0)
    wait_wb(GNCH - 1, (GNCH - 1) % 2)


def _gather_sc(T, dst2, src2):
    mesh = plsc.VectorSubcoreMesh(core_axis_name="c", subcore_axis_name="s")
    f = pl.kernel(
        _gather_body,
        mesh=mesh,
        out_type=[jax.ShapeDtypeStruct((EC, WT), jnp.float32),
                  jax.ShapeDtypeStruct((EC, WT), jnp.float32)],
        scratch_types=[pltpu.VMEM((EPW,), jnp.int32),
                       pltpu.VMEM((EPW,), jnp.int32),
                       pltpu.VMEM((2, GCH, WT), jnp.float32),
                       pltpu.VMEM((2, GCH, WT), jnp.float32),
                       pltpu.VMEM_SHARED((N, WT), jnp.float32),
                       pltpu.SemaphoreType.DMA,
                       pltpu.SemaphoreType.DMA,
                       pltpu.SemaphoreType.DMA],
    )
    return f(T, dst2, src2)


SCH = 128
SNCH = (EPW + SCH - 1) // SCH


def _scatter_body(msgs_hbm, idx3_hbm, zeros_hbm, outp_hbm,
                  idxs_v, upd_v, acc_sh, seml, sema):
    c = lax.axis_index("c")
    s = lax.axis_index("s")
    wid = c * NS + s
    row0 = pl.multiple_of(s * RPS, 8)
    pltpu.sync_copy(zeros_hbm.at[pl.ds(row0, RPS)], acc_sh.at[pl.ds(row0, RPS)])
    pltpu.sync_copy(idx3_hbm.at[wid], idxs_v)
    plsc.subcore_barrier()
    base = pl.multiple_of(wid * EPW, 8)

    def start_load(i, b):
        off = base + _s_off(i)
        pltpu.async_copy(msgs_hbm.at[pl.ds(off, SCH)], upd_v.at[b], seml)

    def wait_load(i, b):
        off = base + _s_off(i)
        pltpu.make_async_copy(msgs_hbm.at[pl.ds(off, SCH)], upd_v.at[b], seml).wait()

    def start_add(i, b):
        pltpu.async_copy(upd_v.at[b], acc_sh.at[idxs_v.at[i]], sema, add=True)

    def wait_add(i, b):
        pltpu.make_async_copy(upd_v.at[b], acc_sh.at[idxs_v.at[i]], sema).wait()

    start_load(0, 0)

    def body(i, carry):
        b = lax.rem(i, 2)

        @pl.when(i >= 1)
        def _():
            wait_add(i - 1, lax.rem(i + 1, 2))

        @pl.when(i + 1 < SNCH)
        def _():
            start_load(i + 1, lax.rem(i + 1, 2))

        wait_load(i, b)
        start_add(i, b)
        return carry

    lax.fori_loop(0, SNCH, body, 0)
    wait_add(SNCH - 1, (SNCH - 1) % 2)
    plsc.subcore_barrier()
    pltpu.sync_copy(acc_sh.at[pl.ds(row0, RPS)], outp_hbm.at[c, pl.ds(row0, RPS)])


def _scatter_sc(msgs, idx3, zeros):
    mesh = plsc.VectorSubcoreMesh(core_axis_name="c", subcore_axis_name="s")
    f = pl.kernel(
        _scatter_body,
        mesh=mesh,
        out_type=jax.ShapeDtypeStruct((NC, NP, WO), jnp.float32),
        scratch_types=[pltpu.VMEM((SNCH, SCH), jnp.int32),
                       pltpu.VMEM((2, SCH, WO), jnp.float32),
                       pltpu.VMEM_SHARED((NP, WO), jnp.float32),
                       pltpu.SemaphoreType.DMA,
                       pltpu.SemaphoreType.DMA],
    )
    return f(msgs, idx3, zeros)


def kernel(x, pos, edge_attr, u, W_embed, b_embed, We1, be1, We2, be2,
           Wc1, bc1, Wc2, bc2, Wn1, bn1, Wn2, bn2, W_out, b_out, edge_index):
    src = edge_index[0]
    dst = edge_index[1]
    dst2 = dst.reshape(NCHUNK, NW, EPW)
    src2 = src.reshape(NCHUNK, NW, EPW)
    ea3 = edge_attr.reshape(NCHUNK, EC, ED)
    tpad = SNCH * SCH - EPW
    wid_col = jnp.arange(NW, dtype=jnp.int32)[None, :, None]
    trash = N + (jnp.arange(tpad, dtype=jnp.int32)[None, None, :]
                 + wid_col * 7) % (NP - N)
    trash = jnp.broadcast_to(trash, (NCHUNK, NW, tpad))
    last = jnp.concatenate([trash, dst2[..., (SNCH - 1) * SCH:]], axis=-1)
    idx4 = jnp.concatenate(
        [dst2[..., :(SNCH - 1) * SCH].reshape(NCHUNK, NW, SNCH - 1, SCH),
         last[:, :, None, :]], axis=2)
    zeros = jnp.zeros((NP, WO), jnp.float32)
    T = _embed_call(x, pos, W_embed, b_embed)
    for l in range(L):
        gath = [_gather_sc(T, dst2[k], src2[k]) for k in range(NCHUNK)]
        accs = []
        for k in range(NCHUNK):
            Ti, Tj = gath[k]
            msgs = _edge_call(Ti, Tj, ea3[k], We1[l], be1[l], We2[l], be2[l],
                              Wc1[l], bc1[l], Wc2[l], bc2[l])
            accp = _scatter_sc(msgs, idx4[k], zeros)
            accs.append(accp[0])
            accs.append(accp[1])
        T = _node_call(T, accs, u, Wn1[l], bn1[l], Wn2[l], bn2[l])
    return _head_call(T, W_out, b_out)


# speedup vs baseline: 1.0729x; 1.0729x over previous
import functools

import jax
import jax.numpy as jnp
from jax import lax
from jax.experimental import pallas as pl
from jax.experimental.pallas import tpu as pltpu
from jax.experimental.pallas import tpu_sc as plsc

N = 10000
E = 320000
F_IN = 128
H = 64
ED = 4
L = 4
OUT = 5

WT = 128
WO = 128

BN = 2000
BE = 4000


def _silu(v):
    return v / (1.0 + jnp.exp(-v))


def _embed_body(x_ref, p_ref, w_ref, b_ref, t_ref):
    h = jnp.dot(x_ref[...], w_ref[...], preferred_element_type=jnp.float32)
    h = h + b_ref[...]
    p = p_ref[...]
    p2 = jnp.sum(p * p, axis=1, keepdims=True)
    z = jnp.zeros((h.shape[0], WT - H - 3), jnp.float32)
    t_ref[...] = jnp.concatenate([h, p, p2, z], axis=1)


def _embed_call(x, pos, W_embed, b_embed):
    return pl.pallas_call(
        _embed_body,
        grid=(N // BN,),
        in_specs=[
            pl.BlockSpec((BN, F_IN), lambda i: (i, 0)),
            pl.BlockSpec((BN, 2), lambda i: (i, 0)),
            pl.BlockSpec((F_IN, H), lambda i: (0, 0)),
            pl.BlockSpec((1, H), lambda i: (0, 0)),
        ],
        out_specs=pl.BlockSpec((BN, WT), lambda i: (i, 0)),
        out_shape=jax.ShapeDtypeStruct((N, WT), jnp.float32),
    )(x, pos, W_embed, b_embed.reshape(1, H))


def _edge_body(ti_ref, tj_ref, ea_ref, wa_ref, wb_ref, vb_ref, wc_ref,
               be1_ref, we2_ref, be2_ref, wc1_ref, bc1_ref, wc28_ref, bc2_ref,
               out_ref):
    ti = ti_ref[...]
    tj = tj_ref[...]
    qi = ti[:, H:H + 8]
    qj = tj[:, H:H + 8]
    qq = qi * qj

    pre = jnp.dot(ti, wa_ref[...], preferred_element_type=jnp.float32)
    pre = pre + jnp.dot(tj, wb_ref[...], preferred_element_type=jnp.float32)
    pre = pre + jnp.dot(qq, vb_ref[...], preferred_element_type=jnp.float32)
    pre = pre + jnp.dot(ea_ref[...], wc_ref[...], preferred_element_type=jnp.float32)
    pre = pre + be1_ref[...]
    m = _silu(pre)
    m = _silu(jnp.dot(m, we2_ref[...], preferred_element_type=jnp.float32) + be2_ref[...])
    c1 = _silu(jnp.dot(m, wc1_ref[...], preferred_element_type=jnp.float32) + bc1_ref[...])
    cw8 = jnp.dot(c1, wc28_ref[...], preferred_element_type=jnp.float32) + bc2_ref[0, 0]

    out_ref[:, :H] = m
    out_ref[:, H:H + 8] = (qi - qj) * cw8
    out_ref[:, H + 8:H + 16] = jnp.ones((ti.shape[0], 8), jnp.float32)


def _edge_call(Ti, Tj, edge_attr, We1, be1, We2, be2, Wc1, bc1, Wc2, bc2):
    wd = We1[2 * H:2 * H + 1]
    z2 = jnp.zeros((2, H), jnp.float32)
    z61 = jnp.zeros((WT - H - 3, H), jnp.float32)
    wa = jnp.concatenate([We1[:H], z2, wd, z61], axis=0)
    wb = jnp.concatenate([We1[H:2 * H], z2, wd, z61], axis=0)
    vb = jnp.concatenate([-2.0 * wd, -2.0 * wd,
                          jnp.zeros((6, H), jnp.float32)], axis=0)
    wc = We1[2 * H + 1:]
    wc28 = jnp.tile(Wc2, (1, 8))
    full = lambda shape: pl.BlockSpec(shape, lambda i: (0, 0))
    return pl.pallas_call(
        _edge_body,
        grid=(EC // BE,),
        in_specs=[
            pl.BlockSpec((BE, WT), lambda i: (i, 0)),
            pl.BlockSpec((BE, WT), lambda i: (i, 0)),
            pl.BlockSpec((BE, ED), lambda i: (i, 0)),
            full((WT, H)), full((WT, H)), full((8, H)), full((ED, H)),
            full((1, H)), full((H, H)), full((1, H)),
            full((H, H)), full((1, H)), full((H, 8)), full((1, 1)),
        ],
        out_specs=pl.BlockSpec((BE, WO), lambda i: (i, 0)),
        out_shape=jax.ShapeDtypeStruct((EC, WO), jnp.float32),
    )(Ti, Tj, edge_attr, wa, wb, vb, wc, be1.reshape(1, H), We2,
      be2.reshape(1, H), Wc1, bc1.reshape(1, H), wc28, bc2.reshape(1, 1))


def _node_body(t_ref, a0_ref, a1_ref, a2_ref, a3_ref, u_ref, wn1a_ref,
               wn1b_ref, wn1c_ref, bn1_ref, wn2_ref, bn2_ref, tout_ref):
    t = t_ref[...]
    h = t[:, :H]
    p = t[:, H:H + 2]
    acc = (a0_ref[...] + a1_ref[...]) + (a2_ref[...] + a3_ref[...])
    agg = acc[:, :H]
    sp = acc[:, H:H + 2]
    deg = acc[:, H + 8:H + 9]
    p_new = p + sp / (deg + 1.0)

    ub = u_ref[0, 0] * wn1c_ref[0:1, :] + u_ref[0, 1] * wn1c_ref[1:2, :]
    pre = jnp.dot(h, wn1a_ref[...], preferred_element_type=jnp.float32)
    pre = pre + jnp.dot(agg, wn1b_ref[...], preferred_element_type=jnp.float32)
    pre = pre + ub + bn1_ref[...]
    hn = _silu(pre)
    hn = jnp.dot(hn, wn2_ref[...], preferred_element_type=jnp.float32) + bn2_ref[...]
    h_new = h + hn
    p2 = jnp.sum(p_new * p_new, axis=1, keepdims=True)
    z = jnp.zeros((h_new.shape[0], WT - H - 3), jnp.float32)
    tout_ref[...] = jnp.concatenate([h_new, p_new, p2, z], axis=1)


def _node_call(T, accs, u, Wn1, bn1, Wn2, bn2):
    wn1a = Wn1[:H]
    wn1b = Wn1[H:2 * H]
    wn1c = Wn1[2 * H:]
    full = lambda shape: pl.BlockSpec(shape, lambda i: (0, 0))
    acc_spec = pl.BlockSpec((BN, WO), lambda i: (i, 0))
    return pl.pallas_call(
        _node_body,
        grid=(N // BN,),
        in_specs=[
            pl.BlockSpec((BN, WT), lambda i: (i, 0)),
            acc_spec, acc_spec, acc_spec, acc_spec,
            full((1, 2)),
            full((H, H)), full((H, H)), full((2, H)),
            full((1, H)), full((H, H)), full((1, H)),
        ],
        out_specs=pl.BlockSpec((BN, WT), lambda i: (i, 0)),
        out_shape=jax.ShapeDtypeStruct((N, WT), jnp.float32),
    )(T, accs[0], accs[1], accs[2], accs[3], u, wn1a, wn1b, wn1c,
      bn1.reshape(1, H), Wn2, bn2.reshape(1, H))


def _head_body(t_ref, w_ref, b_ref, o_ref):
    h = t_ref[:, :H]
    o_ref[...] = jnp.dot(h, w_ref[...], preferred_element_type=jnp.float32) + b_ref[...]


def _head_call(T, W_out, b_out):
    return pl.pallas_call(
        _head_body,
        grid=(N // BN,),
        in_specs=[
            pl.BlockSpec((BN, WT), lambda i: (i, 0)),
            pl.BlockSpec((H, OUT), lambda i: (0, 0)),
            pl.BlockSpec((1, OUT), lambda i: (0, 0)),
        ],
        out_specs=pl.BlockSpec((BN, OUT), lambda i: (i, 0)),
        out_shape=jax.ShapeDtypeStruct((N, OUT), jnp.float32),
    )(T, W_out, b_out.reshape(1, OUT))


NC = 2
NS = 16
NW = NC * NS
NCHUNK = 2
EC = E // NCHUNK
EPW = EC // NW
GCH = 64
GNCH = (EPW + GCH - 1) // GCH
TSR = 640
NP = 10240
RPS = NP // NS


def _chunk_off(i):
    return pl.multiple_of(jnp.minimum(i * GCH, EPW - GCH), 8)


def _s_off(i):
    return pl.multiple_of(jnp.minimum(i * SCH, EPW - SCH), 8)


def _gather_body(t_hbm, dst2_hbm, src2_hbm, ti_hbm, tj_hbm,
                 idxd_v, idxs_v, bufd, bufs, tab_sh, semd, sems, semw):
    c = lax.axis_index("c")
    s = lax.axis_index("s")
    wid = s * NC + c
    base = wid * EPW
    r0 = pl.multiple_of(jnp.minimum(s * TSR, N - TSR), 8)
    pltpu.sync_copy(t_hbm.at[pl.ds(r0, TSR)], tab_sh.at[pl.ds(r0, TSR)])
    pltpu.sync_copy(dst2_hbm.at[wid], idxd_v)
    pltpu.sync_copy(src2_hbm.at[wid], idxs_v)
    plsc.subcore_barrier()

    def start_gather(i, b):
        off = _chunk_off(i)
        pltpu.async_copy(tab_sh.at[idxd_v.at[pl.ds(off, GCH)]], bufd.at[b], semd)
        pltpu.async_copy(tab_sh.at[idxs_v.at[pl.ds(off, GCH)]], bufs.at[b], sems)

    def wait_gather(i, b):
        off = _chunk_off(i)
        pltpu.make_async_copy(tab_sh.at[idxd_v.at[pl.ds(off, GCH)]], bufd.at[b], semd).wait()
        pltpu.make_async_copy(tab_sh.at[idxs_v.at[pl.ds(off, GCH)]], bufs.at[b], sems).wait()

    def start_wb(i, b):
        off = pl.multiple_of(base, 8) + _chunk_off(i)
        pltpu.async_copy(bufd.at[b], ti_hbm.at[pl.ds(off, GCH)], semw)
        pltpu.async_copy(bufs.at[b], tj_hbm.at[pl.ds(off, GCH)], semw)

    def wait_wb(i, b):
        off = pl.multiple_of(base, 8) + _chunk_off(i)
        pltpu.make_async_copy(bufd.at[b], ti_hbm.at[pl.ds(off, GCH)], semw).wait()
        pltpu.make_async_copy(bufs.at[b], tj_hbm.at[pl.ds(off, GCH)], semw).wait()

    start_gather(0, 0)

    def body(i, carry):
        b = lax.rem(i, 2)

        @pl.when(i >= 1)
        def _():
            wait_wb(i - 1, lax.rem(i + 1, 2))

        @pl.when(i + 1 < GNCH)
        def _():
            start_gather(i + 1, lax.rem(i + 1, 2))

        wait_gather(i, b)
        start_wb(i, b)
        return carry

    lax.fori_loop(0, GNCH, body, 0)
    wait_wb(GNCH - 1, (GNCH - 1) % 2)


def _gather_sc(T, dst2, src2):
    mesh = plsc.VectorSubcoreMesh(core_axis_name="c", subcore_axis_name="s")
    f = pl.kernel(
        _gather_body,
        mesh=mesh,
        out_type=[jax.ShapeDtypeStruct((EC, WT), jnp.float32),
                  jax.ShapeDtypeStruct((EC, WT), jnp.float32)],
        scratch_types=[pltpu.VMEM((EPW,), jnp.int32),
                       pltpu.VMEM((EPW,), jnp.int32),
                       pltpu.VMEM((2, GCH, WT), jnp.float32),
                       pltpu.VMEM((2, GCH, WT), jnp.float32),
                       pltpu.VMEM_SHARED((N, WT), jnp.float32),
                       pltpu.SemaphoreType.DMA,
                       pltpu.SemaphoreType.DMA,
                       pltpu.SemaphoreType.DMA],
    )
    return f(T, dst2, src2)


SCH = 128
SNCH = (EPW + SCH - 1) // SCH


def _scatter_body(msgs_hbm, idx3_hbm, zeros_hbm, outp_hbm,
                  idxs_v, upd_v, acc_sh, seml, sema):
    c = lax.axis_index("c")
    s = lax.axis_index("s")
    wid = c * NS + s
    row0 = pl.multiple_of(s * RPS, 8)
    pltpu.sync_copy(zeros_hbm.at[pl.ds(row0, RPS)], acc_sh.at[pl.ds(row0, RPS)])
    pltpu.sync_copy(idx3_hbm.at[wid], idxs_v)
    plsc.subcore_barrier()
    base = pl.multiple_of(wid * EPW, 8)

    def start_load(i, b):
        off = base + _s_off(i)
        pltpu.async_copy(msgs_hbm.at[pl.ds(off, SCH)], upd_v.at[b], seml)

    def wait_load(i, b):
        off = base + _s_off(i)
        pltpu.make_async_copy(msgs_hbm.at[pl.ds(off, SCH)], upd_v.at[b], seml).wait()

    def start_add(i, b):
        pltpu.async_copy(upd_v.at[b], acc_sh.at[idxs_v.at[i]], sema, add=True)

    def wait_add(i, b):
        pltpu.make_async_copy(upd_v.at[b], acc_sh.at[idxs_v.at[i]], sema).wait()

    start_load(0, 0)

    def body(i, carry):
        b = lax.rem(i, 2)

        @pl.when(i >= 1)
        def _():
            wait_add(i - 1, lax.rem(i + 1, 2))

        @pl.when(i + 1 < SNCH)
        def _():
            start_load(i + 1, lax.rem(i + 1, 2))

        wait_load(i, b)
        start_add(i, b)
        return carry

    lax.fori_loop(0, SNCH, body, 0)
    wait_add(SNCH - 1, (SNCH - 1) % 2)
    plsc.subcore_barrier()
    pltpu.sync_copy(acc_sh.at[pl.ds(row0, RPS)], outp_hbm.at[c, pl.ds(row0, RPS)])


def _scatter_sc(msgs, idx3, zeros):
    mesh = plsc.VectorSubcoreMesh(core_axis_name="c", subcore_axis_name="s")
    f = pl.kernel(
        _scatter_body,
        mesh=mesh,
        out_type=jax.ShapeDtypeStruct((NC, NP, WO), jnp.float32),
        scratch_types=[pltpu.VMEM((SNCH, SCH), jnp.int32),
                       pltpu.VMEM((2, SCH, WO), jnp.float32),
                       pltpu.VMEM_SHARED((NP, WO), jnp.float32),
                       pltpu.SemaphoreType.DMA,
                       pltpu.SemaphoreType.DMA],
    )
    return f(msgs, idx3, zeros)


def kernel(x, pos, edge_attr, u, W_embed, b_embed, We1, be1, We2, be2,
           Wc1, bc1, Wc2, bc2, Wn1, bn1, Wn2, bn2, W_out, b_out, edge_index):
    src = edge_index[0]
    dst = edge_index[1]
    dst2 = dst.reshape(NCHUNK, NW, EPW)
    src2 = src.reshape(NCHUNK, NW, EPW)
    ea3 = edge_attr.reshape(NCHUNK, EC, ED)
    tpad = SNCH * SCH - EPW
    wid_col = jnp.arange(NW, dtype=jnp.int32)[None, :, None]
    trash = N + (jnp.arange(tpad, dtype=jnp.int32)[None, None, :]
                 + wid_col * 7) % (NP - N)
    trash = jnp.broadcast_to(trash, (NCHUNK, NW, tpad))
    last = jnp.concatenate([trash, dst2[..., (SNCH - 1) * SCH:]], axis=-1)
    idx4 = jnp.concatenate(
        [dst2[..., :(SNCH - 1) * SCH].reshape(NCHUNK, NW, SNCH - 1, SCH),
         last[:, :, None, :]], axis=2)
    zeros = jnp.zeros((NP, WO), jnp.float32)
    T = _embed_call(x, pos, W_embed, b_embed)
    for l in range(L):
        gath = [_gather_sc(T, dst2[k], src2[k]) for k in range(NCHUNK)]
        accs = []
        for k in range(NCHUNK):
            Ti, Tj = gath[k]
            msgs = _edge_call(Ti, Tj, ea3[k], We1[l], be1[l], We2[l], be2[l],
                              Wc1[l], bc1[l], Wc2[l], bc2[l])
            accp = _scatter_sc(msgs, idx4[k], zeros)
            accs.append(accp[0])
            accs.append(accp[1])
        T = _node_call(T, accs, u, Wn1[l], bn1[l], Wn2[l], bn2[l])
    return _head_call(T, W_out, b_out)


# speedup vs baseline: 1.0864x; 1.0126x over previous
import functools

import jax
import jax.numpy as jnp
from jax import lax
from jax.experimental import pallas as pl
from jax.experimental.pallas import tpu as pltpu
from jax.experimental.pallas import tpu_sc as plsc

N = 10000
E = 320000
F_IN = 128
H = 64
ED = 4
L = 4
OUT = 5

WT = 128
WO = 128

BN = 2000
BE = 8000


def _silu(v):
    return v / (1.0 + jnp.exp(-v))


def _embed_body(x_ref, p_ref, w_ref, b_ref, t_ref):
    h = jnp.dot(x_ref[...], w_ref[...], preferred_element_type=jnp.float32)
    h = h + b_ref[...]
    p = p_ref[...]
    p2 = jnp.sum(p * p, axis=1, keepdims=True)
    z = jnp.zeros((h.shape[0], WT - H - 3), jnp.float32)
    t_ref[...] = jnp.concatenate([h, p, p2, z], axis=1)


def _embed_call(x, pos, W_embed, b_embed):
    return pl.pallas_call(
        _embed_body,
        grid=(N // BN,),
        in_specs=[
            pl.BlockSpec((BN, F_IN), lambda i: (i, 0)),
            pl.BlockSpec((BN, 2), lambda i: (i, 0)),
            pl.BlockSpec((F_IN, H), lambda i: (0, 0)),
            pl.BlockSpec((1, H), lambda i: (0, 0)),
        ],
        out_specs=pl.BlockSpec((BN, WT), lambda i: (i, 0)),
        out_shape=jax.ShapeDtypeStruct((N, WT), jnp.float32),
    )(x, pos, W_embed, b_embed.reshape(1, H))


def _edge_body(ti_ref, tj_ref, ea_ref, wa_ref, wb_ref, vb_ref, wc_ref,
               be1_ref, we2_ref, be2_ref, wc1_ref, bc1_ref, wc28_ref, bc2_ref,
               out_ref):
    ti = ti_ref[...]
    tj = tj_ref[...]
    qi = ti[:, H:H + 8]
    qj = tj[:, H:H + 8]
    qq = qi * qj

    pre = jnp.dot(ti, wa_ref[...], preferred_element_type=jnp.float32)
    pre = pre + jnp.dot(tj, wb_ref[...], preferred_element_type=jnp.float32)
    pre = pre + jnp.dot(qq, vb_ref[...], preferred_element_type=jnp.float32)
    pre = pre + jnp.dot(ea_ref[...], wc_ref[...], preferred_element_type=jnp.float32)
    pre = pre + be1_ref[...]
    m = _silu(pre)
    m = _silu(jnp.dot(m, we2_ref[...], preferred_element_type=jnp.float32) + be2_ref[...])
    c1 = _silu(jnp.dot(m, wc1_ref[...], preferred_element_type=jnp.float32) + bc1_ref[...])
    cw8 = jnp.dot(c1, wc28_ref[...], preferred_element_type=jnp.float32) + bc2_ref[0, 0]

    out_ref[:, :H] = m
    out_ref[:, H:H + 8] = (qi - qj) * cw8
    out_ref[:, H + 8:H + 16] = jnp.ones((ti.shape[0], 8), jnp.float32)


def _edge_call(Ti, Tj, edge_attr, We1, be1, We2, be2, Wc1, bc1, Wc2, bc2):
    wd = We1[2 * H:2 * H + 1]
    z2 = jnp.zeros((2, H), jnp.float32)
    z61 = jnp.zeros((WT - H - 3, H), jnp.float32)
    wa = jnp.concatenate([We1[:H], z2, wd, z61], axis=0)
    wb = jnp.concatenate([We1[H:2 * H], z2, wd, z61], axis=0)
    vb = jnp.concatenate([-2.0 * wd, -2.0 * wd,
                          jnp.zeros((6, H), jnp.float32)], axis=0)
    wc = We1[2 * H + 1:]
    wc28 = jnp.tile(Wc2, (1, 8))
    full = lambda shape: pl.BlockSpec(shape, lambda i: (0, 0))
    return pl.pallas_call(
        _edge_body,
        grid=(EC // BE,),
        in_specs=[
            pl.BlockSpec((BE, WT), lambda i: (i, 0)),
            pl.BlockSpec((BE, WT), lambda i: (i, 0)),
            pl.BlockSpec((BE, ED), lambda i: (i, 0)),
            full((WT, H)), full((WT, H)), full((8, H)), full((ED, H)),
            full((1, H)), full((H, H)), full((1, H)),
            full((H, H)), full((1, H)), full((H, 8)), full((1, 1)),
        ],
        out_specs=pl.BlockSpec((BE, WO), lambda i: (i, 0)),
        out_shape=jax.ShapeDtypeStruct((EC, WO), jnp.float32),
    )(Ti, Tj, edge_attr, wa, wb, vb, wc, be1.reshape(1, H), We2,
      be2.reshape(1, H), Wc1, bc1.reshape(1, H), wc28, bc2.reshape(1, 1))


def _node_body(t_ref, a0_ref, a1_ref, a2_ref, a3_ref, u_ref, wn1a_ref,
               wn1b_ref, wn1c_ref, bn1_ref, wn2_ref, bn2_ref, tout_ref):
    t = t_ref[...]
    h = t[:, :H]
    p = t[:, H:H + 2]
    acc = (a0_ref[...] + a1_ref[...]) + (a2_ref[...] + a3_ref[...])
    agg = acc[:, :H]
    sp = acc[:, H:H + 2]
    deg = acc[:, H + 8:H + 9]
    p_new = p + sp / (deg + 1.0)

    ub = u_ref[0, 0] * wn1c_ref[0:1, :] + u_ref[0, 1] * wn1c_ref[1:2, :]
    pre = jnp.dot(h, wn1a_ref[...], preferred_element_type=jnp.float32)
    pre = pre + jnp.dot(agg, wn1b_ref[...], preferred_element_type=jnp.float32)
    pre = pre + ub + bn1_ref[...]
    hn = _silu(pre)
    hn = jnp.dot(hn, wn2_ref[...], preferred_element_type=jnp.float32) + bn2_ref[...]
    h_new = h + hn
    p2 = jnp.sum(p_new * p_new, axis=1, keepdims=True)
    z = jnp.zeros((h_new.shape[0], WT - H - 3), jnp.float32)
    tout_ref[...] = jnp.concatenate([h_new, p_new, p2, z], axis=1)


def _node_call(T, accs, u, Wn1, bn1, Wn2, bn2):
    wn1a = Wn1[:H]
    wn1b = Wn1[H:2 * H]
    wn1c = Wn1[2 * H:]
    full = lambda shape: pl.BlockSpec(shape, lambda i: (0, 0))
    acc_spec = pl.BlockSpec((BN, WO), lambda i: (i, 0))
    return pl.pallas_call(
        _node_body,
        grid=(N // BN,),
        in_specs=[
            pl.BlockSpec((BN, WT), lambda i: (i, 0)),
            acc_spec, acc_spec, acc_spec, acc_spec,
            full((1, 2)),
            full((H, H)), full((H, H)), full((2, H)),
            full((1, H)), full((H, H)), full((1, H)),
        ],
        out_specs=pl.BlockSpec((BN, WT), lambda i: (i, 0)),
        out_shape=jax.ShapeDtypeStruct((N, WT), jnp.float32),
    )(T, accs[0], accs[1], accs[2], accs[3], u, wn1a, wn1b, wn1c,
      bn1.reshape(1, H), Wn2, bn2.reshape(1, H))


def _head_body(t_ref, w_ref, b_ref, o_ref):
    h = t_ref[:, :H]
    o_ref[...] = jnp.dot(h, w_ref[...], preferred_element_type=jnp.float32) + b_ref[...]


def _head_call(T, W_out, b_out):
    return pl.pallas_call(
        _head_body,
        grid=(N // BN,),
        in_specs=[
            pl.BlockSpec((BN, WT), lambda i: (i, 0)),
            pl.BlockSpec((H, OUT), lambda i: (0, 0)),
            pl.BlockSpec((1, OUT), lambda i: (0, 0)),
        ],
        out_specs=pl.BlockSpec((BN, OUT), lambda i: (i, 0)),
        out_shape=jax.ShapeDtypeStruct((N, OUT), jnp.float32),
    )(T, W_out, b_out.reshape(1, OUT))


NC = 2
NS = 16
NW = NC * NS
NCHUNK = 2
EC = E // NCHUNK
EPW = EC // NW
GCH = 64
GNCH = (EPW + GCH - 1) // GCH
TSR = 640
NP = 10240
RPS = NP // NS


def _chunk_off(i):
    return pl.multiple_of(jnp.minimum(i * GCH, EPW - GCH), 8)


def _s_off(i):
    return pl.multiple_of(jnp.minimum(i * SCH, EPW - SCH), 8)


def _gather_body(t_hbm, dst2_hbm, src2_hbm, ti_hbm, tj_hbm,
                 idxd_v, idxs_v, bufd, bufs, tab_sh, semd, sems, semw):
    c = lax.axis_index("c")
    s = lax.axis_index("s")
    wid = s * NC + c
    base = wid * EPW
    r0 = pl.multiple_of(jnp.minimum(s * TSR, N - TSR), 8)
    pltpu.sync_copy(t_hbm.at[pl.ds(r0, TSR)], tab_sh.at[pl.ds(r0, TSR)])
    pltpu.sync_copy(dst2_hbm.at[wid], idxd_v)
    pltpu.sync_copy(src2_hbm.at[wid], idxs_v)
    plsc.subcore_barrier()

    def start_gather(i, b):
        off = _chunk_off(i)
        pltpu.async_copy(tab_sh.at[idxd_v.at[pl.ds(off, GCH)]], bufd.at[b], semd)
        pltpu.async_copy(tab_sh.at[idxs_v.at[pl.ds(off, GCH)]], bufs.at[b], sems)

    def wait_gather(i, b):
        off = _chunk_off(i)
        pltpu.make_async_copy(tab_sh.at[idxd_v.at[pl.ds(off, GCH)]], bufd.at[b], semd).wait()
        pltpu.make_async_copy(tab_sh.at[idxs_v.at[pl.ds(off, GCH)]], bufs.at[b], sems).wait()

    def start_wb(i, b):
        off = pl.multiple_of(base, 8) + _chunk_off(i)
        pltpu.async_copy(bufd.at[b], ti_hbm.at[pl.ds(off, GCH)], semw)
        pltpu.async_copy(bufs.at[b], tj_hbm.at[pl.ds(off, GCH)], semw)

    def wait_wb(i, b):
        off = pl.multiple_of(base, 8) + _chunk_off(i)
        pltpu.make_async_copy(bufd.at[b], ti_hbm.at[pl.ds(off, GCH)], semw).wait()
        pltpu.make_async_copy(bufs.at[b], tj_hbm.at[pl.ds(off, GCH)], semw).wait()

    start_gather(0, 0)

    def body(i, carry):
        b = lax.rem(i, 2)

        @pl.when(i >= 1)
        def _():
            wait_wb(i - 1, lax.rem(i + 1, 2))

        @pl.when(i + 1 < GNCH)
        def _():
            start_gather(i + 1, lax.rem(i + 1, 2))

        wait_gather(i, b)
        start_wb(i, b)
        return carry

    lax.fori_loop(0, GNCH, body, 0)
    wait_wb(GNCH - 1, (GNCH - 1) % 2)


def _gather_sc(T, dst2, src2):
    mesh = plsc.VectorSubcoreMesh(core_axis_name="c", subcore_axis_name="s")
    f = pl.kernel(
        _gather_body,
        mesh=mesh,
        out_type=[jax.ShapeDtypeStruct((EC, WT), jnp.float32),
                  jax.ShapeDtypeStruct((EC, WT), jnp.float32)],
        scratch_types=[pltpu.VMEM((EPW,), jnp.int32),
                       pltpu.VMEM((EPW,), jnp.int32),
                       pltpu.VMEM((2, GCH, WT), jnp.float32),
                       pltpu.VMEM((2, GCH, WT), jnp.float32),
                       pltpu.VMEM_SHARED((N, WT), jnp.float32),
                       pltpu.SemaphoreType.DMA,
                       pltpu.SemaphoreType.DMA,
                       pltpu.SemaphoreType.DMA],
    )
    return f(T, dst2, src2)


SCH = 128
SNCH = (EPW + SCH - 1) // SCH


def _scatter_body(msgs_hbm, idx3_hbm, zeros_hbm, outp_hbm,
                  idxs_v, upd_v, acc_sh, seml, sema):
    c = lax.axis_index("c")
    s = lax.axis_index("s")
    wid = c * NS + s
    row0 = pl.multiple_of(s * RPS, 8)
    pltpu.sync_copy(zeros_hbm.at[pl.ds(row0, RPS)], acc_sh.at[pl.ds(row0, RPS)])
    pltpu.sync_copy(idx3_hbm.at[wid], idxs_v)
    plsc.subcore_barrier()
    base = pl.multiple_of(wid * EPW, 8)

    def start_load(i, b):
        off = base + _s_off(i)
        pltpu.async_copy(msgs_hbm.at[pl.ds(off, SCH)], upd_v.at[b], seml)

    def wait_load(i, b):
        off = base + _s_off(i)
        pltpu.make_async_copy(msgs_hbm.at[pl.ds(off, SCH)], upd_v.at[b], seml).wait()

    def start_add(i, b):
        pltpu.async_copy(upd_v.at[b], acc_sh.at[idxs_v.at[i]], sema, add=True)

    def wait_add(i, b):
        pltpu.make_async_copy(upd_v.at[b], acc_sh.at[idxs_v.at[i]], sema).wait()

    start_load(0, 0)

    def body(i, carry):
        b = lax.rem(i, 2)

        @pl.when(i >= 1)
        def _():
            wait_add(i - 1, lax.rem(i + 1, 2))

        @pl.when(i + 1 < SNCH)
        def _():
            start_load(i + 1, lax.rem(i + 1, 2))

        wait_load(i, b)
        start_add(i, b)
        return carry

    lax.fori_loop(0, SNCH, body, 0)
    wait_add(SNCH - 1, (SNCH - 1) % 2)
    plsc.subcore_barrier()
    pltpu.sync_copy(acc_sh.at[pl.ds(row0, RPS)], outp_hbm.at[c, pl.ds(row0, RPS)])


def _scatter_sc(msgs, idx3, zeros):
    mesh = plsc.VectorSubcoreMesh(core_axis_name="c", subcore_axis_name="s")
    f = pl.kernel(
        _scatter_body,
        mesh=mesh,
        out_type=jax.ShapeDtypeStruct((NC, NP, WO), jnp.float32),
        scratch_types=[pltpu.VMEM((SNCH, SCH), jnp.int32),
                       pltpu.VMEM((2, SCH, WO), jnp.float32),
                       pltpu.VMEM_SHARED((NP, WO), jnp.float32),
                       pltpu.SemaphoreType.DMA,
                       pltpu.SemaphoreType.DMA],
    )
    return f(msgs, idx3, zeros)


def kernel(x, pos, edge_attr, u, W_embed, b_embed, We1, be1, We2, be2,
           Wc1, bc1, Wc2, bc2, Wn1, bn1, Wn2, bn2, W_out, b_out, edge_index):
    src = edge_index[0]
    dst = edge_index[1]
    dst2 = dst.reshape(NCHUNK, NW, EPW)
    src2 = src.reshape(NCHUNK, NW, EPW)
    ea3 = edge_attr.reshape(NCHUNK, EC, ED)
    tpad = SNCH * SCH - EPW
    wid_col = jnp.arange(NW, dtype=jnp.int32)[None, :, None]
    trash = N + (jnp.arange(tpad, dtype=jnp.int32)[None, None, :]
                 + wid_col * 7) % (NP - N)
    trash = jnp.broadcast_to(trash, (NCHUNK, NW, tpad))
    last = jnp.concatenate([trash, dst2[..., (SNCH - 1) * SCH:]], axis=-1)
    idx4 = jnp.concatenate(
        [dst2[..., :(SNCH - 1) * SCH].reshape(NCHUNK, NW, SNCH - 1, SCH),
         last[:, :, None, :]], axis=2)
    zeros = jnp.zeros((NP, WO), jnp.float32)
    T = _embed_call(x, pos, W_embed, b_embed)
    for l in range(L):
        gath = [_gather_sc(T, dst2[k], src2[k]) for k in range(NCHUNK)]
        accs = []
        for k in range(NCHUNK):
            Ti, Tj = gath[k]
            msgs = _edge_call(Ti, Tj, ea3[k], We1[l], be1[l], We2[l], be2[l],
                              Wc1[l], bc1[l], Wc2[l], bc2[l])
            accp = _scatter_sc(msgs, idx4[k], zeros)
            accs.append(accp[0])
            accs.append(accp[1])
        T = _node_call(T, accs, u, Wn1[l], bn1[l], Wn2[l], bn2[l])
    return _head_call(T, W_out, b_out)
